# baseline (speedup 1.0000x reference)
import functools

import jax
import jax.numpy as jnp
from jax import lax
from jax.experimental import pallas as pl
from jax.experimental.pallas import tpu as pltpu
from jax.experimental.pallas import tpu_sc as plsc

B = 4
N1 = 2048
N2 = 512
DD = 128
DS = 256
HID = 256
DO = 256
KI = 8
KN = 16
NP = B * N1
NPK = NP * KN
EPS = 1e-5
_INF = float("inf")

_NC = 2
_NS = 16
_NW = _NC * _NS
_GC = 128


def _bspec(shape, imap):
    return pl.BlockSpec(shape, imap)


def _f32_bits(x):
    return lax.bitcast_convert_type(x, jnp.int32)


def _rne_top(x):
    u = _f32_bits(x)
    r = u + jnp.int32(0x7FFF) + ((u >> 16) & 1)
    return jnp.bitwise_and(r, jnp.int32(-65536))


def _pack_pair(hi, lo):
    return jnp.bitwise_or(_rne_top(hi),
                          lax.shift_right_logical(_rne_top(lo), 16))


def _unpack_lo(u):
    return lax.bitcast_convert_type(lax.shift_left(u, 16), jnp.float32)


def _unpack_hi(u):
    return lax.bitcast_convert_type(jnp.bitwise_and(u, jnp.int32(-65536)),
                                    jnp.float32)


def _k_dense_mm(x1, Wd, bd):
    R = 512
    nt = NP // R

    def body(x_ref, w_ref, b_ref, y_ref, st_ref):
        y = jnp.dot(x_ref[...], w_ref[...],
                    preferred_element_type=jnp.float32) + b_ref[...]
        y_ref[...] = y
        s = jnp.sum(y, axis=0, keepdims=True)
        ss = jnp.sum(y * y, axis=0, keepdims=True)
        upd = jnp.concatenate([s, ss, jnp.zeros((6, DS), jnp.float32)], axis=0)

        @pl.when(pl.program_id(0) == 0)
        def _():
            st_ref[...] = upd

        @pl.when(pl.program_id(0) != 0)
        def _():
            st_ref[...] += upd

    return pl.pallas_call(
        body,
        grid=(nt,),
        in_specs=[_bspec((R, DD), lambda i: (i, 0)),
                  _bspec((DD, DS), lambda i: (0, 0)),
                  _bspec((1, DS), lambda i: (0, 0))],
        out_specs=[_bspec((R, DS), lambda i: (i, 0)),
                   _bspec((8, DS), lambda i: (0, 0))],
        out_shape=[jax.ShapeDtypeStruct((NP, DS), jnp.float32),
                   jax.ShapeDtypeStruct((8, DS), jnp.float32)],
    )(x1, Wd, bd.reshape(1, DS))


def _k_interp(p1T, p2T, x2):
    T = 256
    nt = N1 // T

    def body(pq_ref, pk_ref, x2_ref, o_ref):
        pq = pq_ref[...]
        pk = pk_ref[...]
        n1 = jnp.sum(pq * pq, axis=0, keepdims=True)
        n2 = jnp.sum(pk * pk, axis=0, keepdims=True)
        one_q = jnp.ones((1, T), jnp.float32)
        one_k = jnp.ones((1, N2), jnp.float32)
        pqa = jnp.concatenate([pq, n1, one_q], axis=0)
        pka = jnp.concatenate([-2.0 * pk, one_k, n2], axis=0)
        d2 = lax.dot_general(pqa, pka, (((0,), (0,)), ((), ())),
                             precision=lax.Precision.HIGHEST,
                             preferred_element_type=jnp.float32)
        dist = jnp.sqrt(jnp.maximum(d2, 0.0) + 1e-12)
        dd = dist
        for _ in range(KI - 1):
            m = jnp.min(dd, axis=1, keepdims=True)
            dd = jnp.where(dd == m, _INF, dd)
        t8 = jnp.min(dd, axis=1, keepdims=True)
        w = jnp.where(dist <= t8, 1.0 / (dist + 1e-8), 0.0)
        w = w / jnp.sum(w, axis=1, keepdims=True)
        o_ref[...] = jnp.dot(w, x2_ref[...],
                             preferred_element_type=jnp.float32)

    return pl.pallas_call(
        body,
        grid=(B, nt),
        in_specs=[_bspec((3, T), lambda b, t: (0, b * nt + t)),
                  _bspec((3, N2), lambda b, t: (0, b)),
                  _bspec((N2, DS), lambda b, t: (b, 0))],
        out_specs=_bspec((T, DS), lambda b, t: (b * nt + t, 0)),
        out_shape=jax.ShapeDtypeStruct((NP, DS), jnp.float32),
    )(p1T, p2T, x2)


def _k_knn(p1T):
    T = 256
    nt = N1 // T

    def body(pq_ref, pa_ref, o_ref):
        pq = pq_ref[...]
        pa = pa_ref[...]
        n1 = jnp.sum(pq * pq, axis=0, keepdims=True)
        na = jnp.sum(pa * pa, axis=0, keepdims=True)
        pqa = jnp.concatenate([pq, n1, jnp.ones((1, T), jnp.float32)], axis=0)
        paa = jnp.concatenate([-2.0 * pa, jnp.ones((1, N1), jnp.float32), na],
                              axis=0)
        d2 = lax.dot_general(pqa, paa, (((0,), (0,)), ((), ())),
                             precision=lax.Precision.HIGHEST,
                             preferred_element_type=jnp.float32)
        iota = lax.broadcasted_iota(jnp.int32, (T, N1), 1).astype(jnp.float32)
        kiota = lax.broadcasted_iota(jnp.int32, (T, KN), 1).astype(jnp.float32)
        acc = jnp.zeros((T, KN), jnp.float32)
        dd = d2
        for k in range(KN):
            m = jnp.min(dd, axis=1, keepdims=True)
            hit = dd == m
            ind = jnp.min(jnp.where(hit, iota, _INF), axis=1, keepdims=True)
            acc = acc + jnp.where(kiota == jnp.float32(k), ind, 0.0)
            dd = jnp.where(hit, _INF, dd)
        o_ref[...] = acc.astype(jnp.int32) + pl.program_id(0) * N1

    return pl.pallas_call(
        body,
        grid=(B, nt),
        in_specs=[_bspec((3, T), lambda b, t: (0, b * nt + t)),
                  _bspec((3, N1), lambda b, t: (0, b))],
        out_specs=_bspec((T, KN), lambda b, t: (b * nt + t, 0)),
        out_shape=jax.ShapeDtypeStruct((NP, KN), jnp.int32),
    )(p1T, p1T)


def _k_derived(y1, st1, g_d, be_d, x_r, Wq, Wk, Wv, W_a1, b_a1):
    R = 512
    nt = NP // R

    def body(y_ref, st_ref, g_ref, be_ref, xr_ref, wq_ref, wk_ref, wv_ref,
             wa1_ref, ba1_ref, a_ref, t2_ref):
        st = st_ref[...]
        mu = st[0:1, :] / NP
        var = st[1:2, :] / NP - mu * mu
        inv = lax.rsqrt(var + EPS)
        xe = jnp.maximum((y_ref[...] - mu) * inv * g_ref[...] + be_ref[...],
                         0.0)
        xr = xr_ref[...]
        q = jnp.dot(xe, wq_ref[...], preferred_element_type=jnp.float32)
        a = jnp.dot(q, wa1_ref[...],
                    preferred_element_type=jnp.float32) + ba1_ref[...]
        kk = jnp.dot(xr, wk_ref[...], preferred_element_type=jnp.float32)
        c = jnp.dot(xe - kk, wa1_ref[...], preferred_element_type=jnp.float32)
        w = jnp.dot(xr, wv_ref[...], preferred_element_type=jnp.float32) + xe
        a_ref[...] = a
        t2_ref[...] = _pack_pair(w, c)

    return pl.pallas_call(
        body,
        grid=(nt,),
        in_specs=[_bspec((R, DS), lambda i: (i, 0)),
                  _bspec((8, DS), lambda i: (0, 0)),
                  _bspec((1, DS), lambda i: (0, 0)),
                  _bspec((1, DS), lambda i: (0, 0)),
                  _bspec((R, DS), lambda i: (i, 0)),
                  _bspec((DS, DS), lambda i: (0, 0)),
                  _bspec((DS, DS), lambda i: (0, 0)),
                  _bspec((DS, DS), lambda i: (0, 0)),
                  _bspec((DS, HID), lambda i: (0, 0)),
                  _bspec((1, HID), lambda i: (0, 0))],
        out_specs=[_bspec((R, HID), lambda i: (i, 0)),
                   _bspec((R, DS), lambda i: (i, 0))],
        out_shape=[jax.ShapeDtypeStruct((NP, HID), jnp.float32),
                   jax.ShapeDtypeStruct((NP, DS), jnp.int32)],
    )(y1, st1, g_d.reshape(1, DS), be_d.reshape(1, DS), x_r, Wq, Wk, Wv,
      W_a1, b_a1.reshape(1, HID))


def _sc_gather(table, idx):
    rows_per_w = NPK // _NW
    GC = 64
    NBUF = 4
    outer = rows_per_w // (GC * NBUF)
    mesh = plsc.VectorSubcoreMesh(core_axis_name="c", subcore_axis_name="s")

    @functools.partial(
        pl.kernel, mesh=mesh,
        out_type=jax.ShapeDtypeStruct((NPK, DS), jnp.int32),
        scratch_types=[pltpu.VMEM((rows_per_w,), jnp.int32)]
                      + [pltpu.VMEM((GC, DS), jnp.int32) for _ in range(NBUF)]
                      + [pltpu.SemaphoreType.DMA for _ in range(2 * NBUF)],
    )
    def gk(table_hbm, idx_hbm, out_hbm, idx_v, r0, r1, r2, r3,
           g0, g1, g2, g3, w0, w1, w2, w3):
        rs = (r0, r1, r2, r3)
        gs = (g0, g1, g2, g3)
        ws = (w0, w1, w2, w3)
        wid = lax.axis_index("s") * _NC + lax.axis_index("c")
        base = wid * rows_per_w
        pltpu.sync_copy(idx_hbm.at[pl.ds(base, rows_per_w)], idx_v)

        def _drain(sem, buf):
            pltpu.make_async_copy(out_hbm.at[pl.ds(0, GC)], buf, sem).wait()

        def body(j, carry):
            for b in range(NBUF):
                @pl.when(j > 0)
                def _(b=b):
                    _drain(ws[b], rs[b])
                pltpu.async_copy(
                    table_hbm.at[idx_v.at[pl.ds((j * NBUF + b) * GC, GC)]],
                    rs[b], gs[b])
            for b in range(NBUF):
                _drain(gs[b], rs[b])
                pltpu.async_copy(
                    rs[b],
                    out_hbm.at[pl.ds(base + (j * NBUF + b) * GC, GC)], ws[b])
            return carry

        lax.fori_loop(0, outer, body, 0)
        for b in range(NBUF):
            _drain(ws[b], rs[b])

    return gk(table, idx)


def _k_pair_stats(G, a):
    P = 128
    nt = NP // P

    def body(g_ref, a_ref, st_ref):
        c3 = _unpack_lo(g_ref[...]).reshape(P, KN, HID)
        z = (c3 + a_ref[...][:, None, :]).reshape(P * KN, HID)
        ones = jnp.ones((8, P * KN), jnp.float32)
        s = jnp.dot(ones, z, preferred_element_type=jnp.float32)
        ss = jnp.dot(ones, z * z, preferred_element_type=jnp.float32)
        upd = jnp.concatenate([s, ss], axis=0)

        @pl.when(pl.program_id(0) == 0)
        def _():
            st_ref[...] = upd

        @pl.when(pl.program_id(0) != 0)
        def _():
            st_ref[...] += upd

    return pl.pallas_call(
        body,
        grid=(nt,),
        in_specs=[_bspec((P * KN, HID), lambda i: (i, 0)),
                  _bspec((P, HID), lambda i: (i, 0))],
        out_specs=_bspec((16, HID), lambda i: (0, 0)),
        out_shape=jax.ShapeDtypeStruct((16, HID), jnp.float32),
    )(G, a)


def _k_attn(G, a, st2, g_a1, be_a1, W_a2, b_a2):
    P = 128
    nt = NP // P

    def body(g_ref, a_ref, st_ref, ga_ref, bea_ref, wa2_ref, ba2_ref, o_ref):
        Gt = g_ref[...]
        st = st_ref[...]
        mu = jnp.sum(st[0:8, :], axis=0, keepdims=True) / (8.0 * NPK)
        var = jnp.sum(st[8:16, :], axis=0, keepdims=True) / (8.0 * NPK) - mu * mu
        invg = lax.rsqrt(var + EPS) * ga_ref[...]
        A = (a_ref[...] - mu) * invg + bea_ref[...]
        cs = _unpack_lo(Gt).reshape(P, KN, HID) * invg.reshape(1, 1, HID)
        w3 = _unpack_hi(Gt).reshape(P, KN, DS)
        h = jnp.maximum(cs + A[:, None, :], 0.0)
        sim = jnp.dot(h.reshape(P * KN, HID).astype(jnp.bfloat16),
                      wa2_ref[...].astype(jnp.bfloat16),
                      preferred_element_type=jnp.float32) + ba2_ref[...]
        e = jnp.exp(sim.reshape(P, KN, DS))
        rec = 1.0 / jnp.sum(e, axis=1, keepdims=True)
        o_ref[...] = jnp.sum(e * w3, axis=1) * rec.reshape(P, DS)

    return pl.pallas_call(
        body,
        grid=(nt,),
        in_specs=[_bspec((P * KN, DS), lambda i: (i, 0)),
                  _bspec((P, HID), lambda i: (i, 0)),
                  _bspec((16, HID), lambda i: (0, 0)),
                  _bspec((1, HID), lambda i: (0, 0)),
                  _bspec((1, HID), lambda i: (0, 0)),
                  _bspec((HID, DS), lambda i: (0, 0)),
                  _bspec((1, DS), lambda i: (0, 0))],
        out_specs=_bspec((P, DS), lambda i: (i, 0)),
        out_shape=jax.ShapeDtypeStruct((NP, DS), jnp.float32),
    )(G, a, st2, g_a1.reshape(1, HID), be_a1.reshape(1, HID), W_a2,
      b_a2.reshape(1, DS))


def _k_mlp(agg, x1, Wt, Wb, bm):
    R = 512
    nt = NP // R

    def body(ag_ref, x1_ref, wt_ref, wb_ref, b_ref, y_ref, st_ref):
        y = (jnp.dot(ag_ref[...], wt_ref[...],
                     preferred_element_type=jnp.float32)
             + jnp.dot(x1_ref[...], wb_ref[...],
                       preferred_element_type=jnp.float32) + b_ref[...])
        y_ref[...] = y
        s = jnp.sum(y, axis=0, keepdims=True)
        ss = jnp.sum(y * y, axis=0, keepdims=True)
        upd = jnp.concatenate([s, ss, jnp.zeros((6, DO), jnp.float32)], axis=0)

        @pl.when(pl.program_id(0) == 0)
        def _():
            st_ref[...] = upd

        @pl.when(pl.program_id(0) != 0)
        def _():
            st_ref[...] += upd

    return pl.pallas_call(
        body,
        grid=(nt,),
        in_specs=[_bspec((R, DS), lambda i: (i, 0)),
                  _bspec((R, DD), lambda i: (i, 0)),
                  _bspec((DS, DO), lambda i: (0, 0)),
                  _bspec((DD, DO), lambda i: (0, 0)),
                  _bspec((1, DO), lambda i: (0, 0))],
        out_specs=[_bspec((R, DO), lambda i: (i, 0)),
                   _bspec((8, DO), lambda i: (0, 0))],
        out_shape=[jax.ShapeDtypeStruct((NP, DO), jnp.float32),
                   jax.ShapeDtypeStruct((8, DO), jnp.float32)],
    )(agg, x1, Wt, Wb, bm.reshape(1, DO))


def _k_norm(y2, st3, g_m, be_m):
    R = 512
    nt = NP // R

    def body(y_ref, st_ref, g_ref, be_ref, o_ref):
        st = st_ref[...]
        mu = st[0:1, :] / NP
        var = st[1:2, :] / NP - mu * mu
        inv = lax.rsqrt(var + EPS)
        o_ref[...] = jnp.maximum(
            (y_ref[...] - mu) * inv * g_ref[...] + be_ref[...], 0.0)

    return pl.pallas_call(
        body,
        grid=(nt,),
        in_specs=[_bspec((R, DO), lambda i: (i, 0)),
                  _bspec((8, DO), lambda i: (0, 0)),
                  _bspec((1, DO), lambda i: (0, 0)),
                  _bspec((1, DO), lambda i: (0, 0))],
        out_specs=_bspec((R, DO), lambda i: (i, 0)),
        out_shape=jax.ShapeDtypeStruct((NP, DO), jnp.float32),
    )(y2, st3, g_m.reshape(1, DO), be_m.reshape(1, DO))


def kernel(p1, x1, o1, p2, x2, o2, batch_size, W_dense, b_dense, g_dense,
           be_dense, Wq, Wk, Wv, W_a1, b_a1, g_a1, be_a1, W_a2, b_a2, W_mlp,
           b_mlp, g_mlp, be_mlp):
    p1T = p1.T
    p2T = p2.T

    y1, st1 = _k_dense_mm(x1, W_dense, b_dense)
    x_r = _k_interp(p1T, p2T, x2)
    idxg = _k_knn(p1T)
    a, T2 = _k_derived(y1, st1, g_dense, be_dense, x_r, Wq, Wk, Wv, W_a1,
                       b_a1)
    G = _sc_gather(T2, idxg.reshape(-1))
    st2 = _k_pair_stats(G, a)
    agg = _k_attn(G, a, st2, g_a1, be_a1, W_a2, b_a2)
    y2, st3 = _k_mlp(agg, x1, W_mlp[:DS], W_mlp[DS:], b_mlp)
    x = _k_norm(y2, st3, g_mlp, be_mlp)
    return (p1, x, o1)


# speedup vs baseline: 1.0499x; 1.0499x over previous
import functools

import jax
import jax.numpy as jnp
from jax import lax
from jax.experimental import pallas as pl
from jax.experimental.pallas import tpu as pltpu
from jax.experimental.pallas import tpu_sc as plsc

B = 4
N1 = 2048
N2 = 512
DD = 128
DS = 256
HID = 256
DO = 256
KI = 8
KN = 16
NP = B * N1
NPK = NP * KN
EPS = 1e-5
_INF = float("inf")

_NC = 2
_NS = 16
_NW = _NC * _NS
_GC = 128


def _bspec(shape, imap):
    return pl.BlockSpec(shape, imap)


def _f32_bits(x):
    return lax.bitcast_convert_type(x, jnp.int32)


def _dot_x3(qa, ka):
    dims = (((0,), (0,)), ((), ()))
    qh = qa.astype(jnp.bfloat16)
    kh = ka.astype(jnp.bfloat16)
    ql = (qa - qh.astype(jnp.float32)).astype(jnp.bfloat16)
    kl = (ka - kh.astype(jnp.float32)).astype(jnp.bfloat16)
    t = lax.dot_general(qh, kh, dims, preferred_element_type=jnp.float32)
    t += lax.dot_general(qh, kl, dims, preferred_element_type=jnp.float32)
    t += lax.dot_general(ql, kh, dims, preferred_element_type=jnp.float32)
    return t


def _rne_top(x):
    u = _f32_bits(x)
    r = u + jnp.int32(0x7FFF) + ((u >> 16) & 1)
    return jnp.bitwise_and(r, jnp.int32(-65536))


def _pack_pair(hi, lo):
    return jnp.bitwise_or(_rne_top(hi),
                          lax.shift_right_logical(_rne_top(lo), 16))


def _unpack_lo(u):
    return lax.bitcast_convert_type(lax.shift_left(u, 16), jnp.float32)


def _unpack_hi(u):
    return lax.bitcast_convert_type(jnp.bitwise_and(u, jnp.int32(-65536)),
                                    jnp.float32)


def _k_dense_mm(x1, Wd, bd):
    R = 512
    nt = NP // R

    def body(x_ref, w_ref, b_ref, y_ref, st_ref):
        y = jnp.dot(x_ref[...], w_ref[...],
                    preferred_element_type=jnp.float32) + b_ref[...]
        y_ref[...] = y
        s = jnp.sum(y, axis=0, keepdims=True)
        ss = jnp.sum(y * y, axis=0, keepdims=True)
        upd = jnp.concatenate([s, ss, jnp.zeros((6, DS), jnp.float32)], axis=0)

        @pl.when(pl.program_id(0) == 0)
        def _():
            st_ref[...] = upd

        @pl.when(pl.program_id(0) != 0)
        def _():
            st_ref[...] += upd

    return pl.pallas_call(
        body,
        grid=(nt,),
        in_specs=[_bspec((R, DD), lambda i: (i, 0)),
                  _bspec((DD, DS), lambda i: (0, 0)),
                  _bspec((1, DS), lambda i: (0, 0))],
        out_specs=[_bspec((R, DS), lambda i: (i, 0)),
                   _bspec((8, DS), lambda i: (0, 0))],
        out_shape=[jax.ShapeDtypeStruct((NP, DS), jnp.float32),
                   jax.ShapeDtypeStruct((8, DS), jnp.float32)],
    )(x1, Wd, bd.reshape(1, DS))


def _k_interp(p1T, p2T, x2):
    T = 256
    nt = N1 // T

    def body(pq_ref, pk_ref, x2_ref, o_ref):
        pq = pq_ref[...]
        pk = pk_ref[...]
        n1 = jnp.sum(pq * pq, axis=0, keepdims=True)
        n2 = jnp.sum(pk * pk, axis=0, keepdims=True)
        one_q = jnp.ones((1, T), jnp.float32)
        one_k = jnp.ones((1, N2), jnp.float32)
        pqa = jnp.concatenate([pq, n1, one_q], axis=0)
        pka = jnp.concatenate([-2.0 * pk, one_k, n2], axis=0)
        d2 = _dot_x3(pqa, pka)
        dist = jnp.sqrt(jnp.maximum(d2, 0.0) + 1e-12)
        dd = dist
        for _ in range(KI - 1):
            m = jnp.min(dd, axis=1, keepdims=True)
            dd = jnp.where(dd == m, _INF, dd)
        t8 = jnp.min(dd, axis=1, keepdims=True)
        w = jnp.where(dist <= t8, 1.0 / (dist + 1e-8), 0.0)
        w = w / jnp.sum(w, axis=1, keepdims=True)
        o_ref[...] = jnp.dot(w, x2_ref[...],
                             preferred_element_type=jnp.float32)

    return pl.pallas_call(
        body,
        grid=(B, nt),
        in_specs=[_bspec((3, T), lambda b, t: (0, b * nt + t)),
                  _bspec((3, N2), lambda b, t: (0, b)),
                  _bspec((N2, DS), lambda b, t: (b, 0))],
        out_specs=_bspec((T, DS), lambda b, t: (b * nt + t, 0)),
        out_shape=jax.ShapeDtypeStruct((NP, DS), jnp.float32),
    )(p1T, p2T, x2)


def _k_knn(p1T):
    T = 256
    nt = N1 // T

    def body(pq_ref, pa_ref, o_ref):
        pq = pq_ref[...]
        pa = pa_ref[...]
        n1 = jnp.sum(pq * pq, axis=0, keepdims=True)
        na = jnp.sum(pa * pa, axis=0, keepdims=True)
        pqa = jnp.concatenate([pq, n1, jnp.ones((1, T), jnp.float32)], axis=0)
        paa = jnp.concatenate([-2.0 * pa, jnp.ones((1, N1), jnp.float32), na],
                              axis=0)
        d2 = _dot_x3(pqa, paa)
        iota = lax.broadcasted_iota(jnp.int32, (T, N1), 1).astype(jnp.float32)
        kiota = lax.broadcasted_iota(jnp.int32, (T, KN), 1).astype(jnp.float32)
        acc = jnp.zeros((T, KN), jnp.float32)
        dd = d2
        for k in range(KN):
            m = jnp.min(dd, axis=1, keepdims=True)
            hit = dd == m
            ind = jnp.min(jnp.where(hit, iota, _INF), axis=1, keepdims=True)
            acc = acc + jnp.where(kiota == jnp.float32(k), ind, 0.0)
            dd = jnp.where(hit, _INF, dd)
        o_ref[...] = acc.astype(jnp.int32) + pl.program_id(0) * N1

    return pl.pallas_call(
        body,
        grid=(B, nt),
        in_specs=[_bspec((3, T), lambda b, t: (0, b * nt + t)),
                  _bspec((3, N1), lambda b, t: (0, b))],
        out_specs=_bspec((T, KN), lambda b, t: (b * nt + t, 0)),
        out_shape=jax.ShapeDtypeStruct((NP, KN), jnp.int32),
    )(p1T, p1T)


def _k_derived(y1, st1, g_d, be_d, x_r, Wq, Wk, Wv, W_a1, b_a1):
    R = 512
    nt = NP // R

    def body(y_ref, st_ref, g_ref, be_ref, xr_ref, wq_ref, wk_ref, wv_ref,
             wa1_ref, ba1_ref, a_ref, t2_ref):
        st = st_ref[...]
        mu = st[0:1, :] / NP
        var = st[1:2, :] / NP - mu * mu
        inv = lax.rsqrt(var + EPS)
        xe = jnp.maximum((y_ref[...] - mu) * inv * g_ref[...] + be_ref[...],
                         0.0)
        xr = xr_ref[...]
        q = jnp.dot(xe, wq_ref[...], preferred_element_type=jnp.float32)
        a = jnp.dot(q, wa1_ref[...],
                    preferred_element_type=jnp.float32) + ba1_ref[...]
        kk = jnp.dot(xr, wk_ref[...], preferred_element_type=jnp.float32)
        c = jnp.dot(xe - kk, wa1_ref[...], preferred_element_type=jnp.float32)
        w = jnp.dot(xr, wv_ref[...], preferred_element_type=jnp.float32) + xe
        a_ref[...] = a
        t2_ref[...] = _pack_pair(w, c)

    return pl.pallas_call(
        body,
        grid=(nt,),
        in_specs=[_bspec((R, DS), lambda i: (i, 0)),
                  _bspec((8, DS), lambda i: (0, 0)),
                  _bspec((1, DS), lambda i: (0, 0)),
                  _bspec((1, DS), lambda i: (0, 0)),
                  _bspec((R, DS), lambda i: (i, 0)),
                  _bspec((DS, DS), lambda i: (0, 0)),
                  _bspec((DS, DS), lambda i: (0, 0)),
                  _bspec((DS, DS), lambda i: (0, 0)),
                  _bspec((DS, HID), lambda i: (0, 0)),
                  _bspec((1, HID), lambda i: (0, 0))],
        out_specs=[_bspec((R, HID), lambda i: (i, 0)),
                   _bspec((R, DS), lambda i: (i, 0))],
        out_shape=[jax.ShapeDtypeStruct((NP, HID), jnp.float32),
                   jax.ShapeDtypeStruct((NP, DS), jnp.int32)],
    )(y1, st1, g_d.reshape(1, DS), be_d.reshape(1, DS), x_r, Wq, Wk, Wv,
      W_a1, b_a1.reshape(1, HID))


def _sc_gather(table, idx):
    rows_per_w = NPK // _NW
    GC = 64
    NBUF = 4
    outer = rows_per_w // (GC * NBUF)
    mesh = plsc.VectorSubcoreMesh(core_axis_name="c", subcore_axis_name="s")

    @functools.partial(
        pl.kernel, mesh=mesh,
        out_type=jax.ShapeDtypeStruct((NPK, DS), jnp.int32),
        scratch_types=[pltpu.VMEM((rows_per_w,), jnp.int32)]
                      + [pltpu.VMEM((GC, DS), jnp.int32) for _ in range(NBUF)]
                      + [pltpu.SemaphoreType.DMA for _ in range(2 * NBUF)],
    )
    def gk(table_hbm, idx_hbm, out_hbm, idx_v, r0, r1, r2, r3,
           g0, g1, g2, g3, w0, w1, w2, w3):
        rs = (r0, r1, r2, r3)
        gs = (g0, g1, g2, g3)
        ws = (w0, w1, w2, w3)
        wid = lax.axis_index("s") * _NC + lax.axis_index("c")
        base = wid * rows_per_w
        pltpu.sync_copy(idx_hbm.at[pl.ds(base, rows_per_w)], idx_v)

        def _drain(sem, buf):
            pltpu.make_async_copy(out_hbm.at[pl.ds(0, GC)], buf, sem).wait()

        def body(j, carry):
            for b in range(NBUF):
                @pl.when(j > 0)
                def _(b=b):
                    _drain(ws[b], rs[b])
                pltpu.async_copy(
                    table_hbm.at[idx_v.at[pl.ds((j * NBUF + b) * GC, GC)]],
                    rs[b], gs[b])
            for b in range(NBUF):
                _drain(gs[b], rs[b])
                pltpu.async_copy(
                    rs[b],
                    out_hbm.at[pl.ds(base + (j * NBUF + b) * GC, GC)], ws[b])
            return carry

        lax.fori_loop(0, outer, body, 0)
        for b in range(NBUF):
            _drain(ws[b], rs[b])

    return gk(table, idx)


def _k_pair_stats(G, a):
    P = 128
    nt = NP // P

    def body(g_ref, a_ref, st_ref):
        c3 = _unpack_lo(g_ref[...]).reshape(P, KN, HID)
        z = (c3 + a_ref[...][:, None, :]).reshape(P * KN, HID)
        ones = jnp.ones((8, P * KN), jnp.float32)
        s = jnp.dot(ones, z, preferred_element_type=jnp.float32)
        ss = jnp.dot(ones, z * z, preferred_element_type=jnp.float32)
        upd = jnp.concatenate([s, ss], axis=0)

        @pl.when(pl.program_id(0) == 0)
        def _():
            st_ref[...] = upd

        @pl.when(pl.program_id(0) != 0)
        def _():
            st_ref[...] += upd

    return pl.pallas_call(
        body,
        grid=(nt,),
        in_specs=[_bspec((P * KN, HID), lambda i: (i, 0)),
                  _bspec((P, HID), lambda i: (i, 0))],
        out_specs=_bspec((16, HID), lambda i: (0, 0)),
        out_shape=jax.ShapeDtypeStruct((16, HID), jnp.float32),
    )(G, a)


def _k_attn(G, a, st2, g_a1, be_a1, W_a2, b_a2):
    P = 128
    nt = NP // P

    def body(g_ref, a_ref, st_ref, ga_ref, bea_ref, wa2_ref, ba2_ref, o_ref):
        Gt = g_ref[...]
        st = st_ref[...]
        mu = jnp.sum(st[0:8, :], axis=0, keepdims=True) / (8.0 * NPK)
        var = jnp.sum(st[8:16, :], axis=0, keepdims=True) / (8.0 * NPK) - mu * mu
        invg = lax.rsqrt(var + EPS) * ga_ref[...]
        A = (a_ref[...] - mu) * invg + bea_ref[...]
        cs = _unpack_lo(Gt).reshape(P, KN, HID) * invg.reshape(1, 1, HID)
        w3 = _unpack_hi(Gt).reshape(P, KN, DS)
        h = jnp.maximum(cs + A[:, None, :], 0.0)
        sim = jnp.dot(h.reshape(P * KN, HID).astype(jnp.bfloat16),
                      wa2_ref[...].astype(jnp.bfloat16),
                      preferred_element_type=jnp.float32) + ba2_ref[...]
        e = jnp.exp(sim.reshape(P, KN, DS))
        rec = 1.0 / jnp.sum(e, axis=1, keepdims=True)
        o_ref[...] = jnp.sum(e * w3, axis=1) * rec.reshape(P, DS)

    return pl.pallas_call(
        body,
        grid=(nt,),
        in_specs=[_bspec((P * KN, DS), lambda i: (i, 0)),
                  _bspec((P, HID), lambda i: (i, 0)),
                  _bspec((16, HID), lambda i: (0, 0)),
                  _bspec((1, HID), lambda i: (0, 0)),
                  _bspec((1, HID), lambda i: (0, 0)),
                  _bspec((HID, DS), lambda i: (0, 0)),
                  _bspec((1, DS), lambda i: (0, 0))],
        out_specs=_bspec((P, DS), lambda i: (i, 0)),
        out_shape=jax.ShapeDtypeStruct((NP, DS), jnp.float32),
    )(G, a, st2, g_a1.reshape(1, HID), be_a1.reshape(1, HID), W_a2,
      b_a2.reshape(1, DS))


def _k_mlp(agg, x1, Wt, Wb, bm):
    R = 512
    nt = NP // R

    def body(ag_ref, x1_ref, wt_ref, wb_ref, b_ref, y_ref, st_ref):
        y = (jnp.dot(ag_ref[...], wt_ref[...],
                     preferred_element_type=jnp.float32)
             + jnp.dot(x1_ref[...], wb_ref[...],
                       preferred_element_type=jnp.float32) + b_ref[...])
        y_ref[...] = y
        s = jnp.sum(y, axis=0, keepdims=True)
        ss = jnp.sum(y * y, axis=0, keepdims=True)
        upd = jnp.concatenate([s, ss, jnp.zeros((6, DO), jnp.float32)], axis=0)

        @pl.when(pl.program_id(0) == 0)
        def _():
            st_ref[...] = upd

        @pl.when(pl.program_id(0) != 0)
        def _():
            st_ref[...] += upd

    return pl.pallas_call(
        body,
        grid=(nt,),
        in_specs=[_bspec((R, DS), lambda i: (i, 0)),
                  _bspec((R, DD), lambda i: (i, 0)),
                  _bspec((DS, DO), lambda i: (0, 0)),
                  _bspec((DD, DO), lambda i: (0, 0)),
                  _bspec((1, DO), lambda i: (0, 0))],
        out_specs=[_bspec((R, DO), lambda i: (i, 0)),
                   _bspec((8, DO), lambda i: (0, 0))],
        out_shape=[jax.ShapeDtypeStruct((NP, DO), jnp.float32),
                   jax.ShapeDtypeStruct((8, DO), jnp.float32)],
    )(agg, x1, Wt, Wb, bm.reshape(1, DO))


def _k_norm(y2, st3, g_m, be_m):
    R = 512
    nt = NP // R

    def body(y_ref, st_ref, g_ref, be_ref, o_ref):
        st = st_ref[...]
        mu = st[0:1, :] / NP
        var = st[1:2, :] / NP - mu * mu
        inv = lax.rsqrt(var + EPS)
        o_ref[...] = jnp.maximum(
            (y_ref[...] - mu) * inv * g_ref[...] + be_ref[...], 0.0)

    return pl.pallas_call(
        body,
        grid=(nt,),
        in_specs=[_bspec((R, DO), lambda i: (i, 0)),
                  _bspec((8, DO), lambda i: (0, 0)),
                  _bspec((1, DO), lambda i: (0, 0)),
                  _bspec((1, DO), lambda i: (0, 0))],
        out_specs=_bspec((R, DO), lambda i: (i, 0)),
        out_shape=jax.ShapeDtypeStruct((NP, DO), jnp.float32),
    )(y2, st3, g_m.reshape(1, DO), be_m.reshape(1, DO))


def kernel(p1, x1, o1, p2, x2, o2, batch_size, W_dense, b_dense, g_dense,
           be_dense, Wq, Wk, Wv, W_a1, b_a1, g_a1, be_a1, W_a2, b_a2, W_mlp,
           b_mlp, g_mlp, be_mlp):
    p1T = p1.T
    p2T = p2.T

    y1, st1 = _k_dense_mm(x1, W_dense, b_dense)
    x_r = _k_interp(p1T, p2T, x2)
    idxg = _k_knn(p1T)
    a, T2 = _k_derived(y1, st1, g_dense, be_dense, x_r, Wq, Wk, Wv, W_a1,
                       b_a1)
    G = _sc_gather(T2, idxg.reshape(-1))
    st2 = _k_pair_stats(G, a)
    agg = _k_attn(G, a, st2, g_a1, be_a1, W_a2, b_a2)
    y2, st3 = _k_mlp(agg, x1, W_mlp[:DS], W_mlp[DS:], b_mlp)
    x = _k_norm(y2, st3, g_mlp, be_mlp)
    return (p1, x, o1)
